# baseline (speedup 1.0000x reference)
import functools

import jax
import jax.numpy as jnp
from jax import lax
from jax.experimental import pallas as pl
from jax.experimental.pallas import tpu as pltpu
from jax.experimental.pallas import tpu_sc as plsc

N_NODES = 10000
N_EDGES = 320000
D = 128

NC = 2
NS = 16
B = 128
BLK = 4
MCHT = 160
MC0 = 108
MC1 = MCHT - MC0
NBLK0 = MC0 // BLK
NBLK1 = MC1 // BLK
E_PAD = NS * MCHT * B
N_PAD = 10240
RPS = N_PAD // NS
ZCH = 128
CW = 8


def _sc_agg_body(with_count, *refs):
    if with_count:
        (x_hbm, src_hbm, dst_hbm, z128_hbm, z8_hbm, ones_hbm,
         part_hbm, cntp_hbm,
         agg_sh, cnt_sh, srcb, dstb, rows_v, ones_v, sg0, si0) = refs
    else:
        (x_hbm, src_hbm, dst_hbm, z128_hbm,
         part_hbm,
         agg_sh, srcb, dstb, rows_v, ones_v, sg0, si0) = refs
    sg = (sg0,)
    si = (si0,)

    c = lax.axis_index("c")
    s = lax.axis_index("s")
    cbase = lax.select(c == 0, 0, MC0)
    nblk = lax.select(c == 0, NBLK0, NBLK1)

    pltpu.sync_copy(z128_hbm, rows_v)
    for k in range(RPS // ZCH):
        pltpu.sync_copy(rows_v, agg_sh.at[pl.ds(s * RPS + k * ZCH, ZCH)])
    if with_count:
        pltpu.sync_copy(z8_hbm, ones_v)
        for k in range(RPS // ZCH):
            pltpu.sync_copy(ones_v, cnt_sh.at[pl.ds(s * RPS + k * ZCH, ZCH)])
        pltpu.sync_copy(ones_hbm, ones_v)

    plsc.subcore_barrier()

    def block(t, carry):
        tb = lax.rem(t, 2)
        tbn = lax.rem(t + 1, 2)
        pltpu.make_async_copy(
            src_hbm.at[s, pl.ds(cbase + t * BLK, BLK)], srcb.at[tb],
            si[0]).wait()
        pltpu.make_async_copy(
            dst_hbm.at[s, pl.ds(cbase + t * BLK, BLK)], dstb.at[tb],
            si[0]).wait()

        @pl.when(t < nblk - 1)
        def _():
            pltpu.async_copy(
                src_hbm.at[s, pl.ds(cbase + (t + 1) * BLK, BLK)],
                srcb.at[tbn], si[0])
            pltpu.async_copy(
                dst_hbm.at[s, pl.ds(cbase + (t + 1) * BLK, BLK)],
                dstb.at[tbn], si[0])

        for k in range(BLK):
            pltpu.async_copy(x_hbm.at[srcb.at[tb, k]], rows_v, sg[0]).wait()
            pltpu.sync_copy(rows_v, agg_sh.at[dstb.at[tb, k]], add=True)
            if with_count:
                pltpu.sync_copy(ones_v, cnt_sh.at[dstb.at[tb, k]], add=True)
        return carry

    pltpu.async_copy(src_hbm.at[s, pl.ds(cbase, BLK)], srcb.at[0], si[0])
    pltpu.async_copy(dst_hbm.at[s, pl.ds(cbase, BLK)], dstb.at[0], si[0])
    lax.fori_loop(0, nblk, block, 0)

    plsc.subcore_barrier()

    for k in range(RPS // ZCH):
        r0 = s * RPS + k * ZCH
        pltpu.sync_copy(agg_sh.at[pl.ds(r0, ZCH)], rows_v)
        pltpu.sync_copy(rows_v, part_hbm.at[c, pl.ds(r0, ZCH)])
        if with_count:
            pltpu.sync_copy(cnt_sh.at[pl.ds(r0, ZCH)], ones_v)
            pltpu.sync_copy(ones_v, cntp_hbm.at[c, pl.ds(r0, ZCH)])


def _make_sc_agg(with_count):
    mesh = plsc.VectorSubcoreMesh(
        core_axis_name="c", subcore_axis_name="s",
        num_cores=NC, num_subcores=NS)
    sems = [pltpu.SemaphoreType.DMA] * 2
    if with_count:
        out_type = (
            jax.ShapeDtypeStruct((NC, N_PAD, D), jnp.float32),
            jax.ShapeDtypeStruct((NC, N_PAD, CW), jnp.float32),
        )
        scratch = [
            pltpu.VMEM_SHARED((N_PAD, D), jnp.float32),
            pltpu.VMEM_SHARED((N_PAD, CW), jnp.float32),
            pltpu.VMEM((2, BLK, B), jnp.int32),
            pltpu.VMEM((2, BLK, B), jnp.int32),
            pltpu.VMEM((ZCH, D), jnp.float32),
            pltpu.VMEM((B, CW), jnp.float32),
        ] + sems
    else:
        out_type = jax.ShapeDtypeStruct((NC, N_PAD, D), jnp.float32)
        scratch = [
            pltpu.VMEM_SHARED((N_PAD, D), jnp.float32),
            pltpu.VMEM((2, BLK, B), jnp.int32),
            pltpu.VMEM((2, BLK, B), jnp.int32),
            pltpu.VMEM((ZCH, D), jnp.float32),
            pltpu.VMEM((B, CW), jnp.float32),
        ] + sems
    return pl.kernel(
        functools.partial(_sc_agg_body, with_count),
        out_type=out_type, mesh=mesh, scratch_types=scratch,
        compiler_params=pltpu.CompilerParams(use_tc_tiling_on_sc=False),
        name=f"sage_sc_agg_cnt{int(with_count)}")


_R = 1000


def _tc_dense_body(p0, p1, c0, c1, x, wl, wr, b, o):
    agg = p0[...] + p1[...]
    cnt = c0[:, 0:1] + c1[:, 0:1]
    inv = 1.0 / jnp.maximum(cnt, 1.0)
    g = jnp.dot(agg, wl[...], preferred_element_type=jnp.float32)
    h = jnp.dot(x[...], wr[...], preferred_element_type=jnp.float32)
    o[...] = jnp.maximum(g * inv + h + b[...], 0.0)


def _tc_dense(part, cntp, x, wl, wr, b):
    grid = (N_NODES // _R,)
    row = pl.BlockSpec((_R, D), lambda i: (i, 0))
    cb = pl.BlockSpec((_R, CW), lambda i: (i, 0))
    full = pl.BlockSpec((D, D), lambda i: (0, 0))
    bias = pl.BlockSpec((1, D), lambda i: (0, 0))
    return pl.pallas_call(
        _tc_dense_body,
        grid=grid,
        in_specs=[row, row, cb, cb, row, full, full, bias],
        out_specs=row,
        out_shape=jax.ShapeDtypeStruct((N_NODES, D), jnp.float32),
    )(part[0], part[1], cntp[0], cntp[1], x, wl, wr, b.reshape(1, D))


def kernel(x, edge_index, Wl1, Wr1, b1, Wl2, Wr2, b2):
    n_extra = E_PAD - N_EDGES
    src = edge_index[0].astype(jnp.int32)
    dst = edge_index[1].astype(jnp.int32)
    src = jnp.concatenate([src, jnp.zeros((n_extra,), jnp.int32)])
    pad_dst = N_NODES + (jnp.arange(n_extra, dtype=jnp.int32) % (N_PAD - N_NODES))
    dst = jnp.concatenate([dst, pad_dst])
    src = src.reshape(NS, MCHT, B)
    dst = dst.reshape(NS, MCHT, B)
    x = x.astype(jnp.float32)
    z128 = jnp.zeros((ZCH, D), jnp.float32)
    z8 = jnp.zeros((ZCH, CW), jnp.float32)
    ones = jnp.ones((B, CW), jnp.float32)

    part1, cntp = _make_sc_agg(True)(x, src, dst, z128, z8, ones)
    h = _tc_dense(part1, cntp, x, Wl1, Wr1, b1)
    part2 = _make_sc_agg(False)(h, src, dst, z128)
    out = _tc_dense(part2, cntp, h, Wl2, Wr2, b2)
    return out


# speedup vs baseline: 1.4752x; 1.4752x over previous
import functools

import jax
import jax.numpy as jnp
from jax import lax
from jax.experimental import pallas as pl
from jax.experimental.pallas import tpu as pltpu
from jax.experimental.pallas import tpu_sc as plsc

N_NODES = 10000
N_EDGES = 320000
D = 128

NC = 2
NS = 16
B = 128
MCHT = 158
MC0_CNT = 103
MC0_PLAIN = 109
E_PAD = NS * MCHT * B
N_PAD = 10240
RPS = N_PAD // NS
ZCH = 128
CHUNKS = [(k * ZCH, ZCH) for k in range(RPS // ZCH)]
CW = 8


def _sc_agg_body(with_count, mc0, *refs):
    if with_count:
        (x_hbm, src_hbm, dst_hbm, z128_hbm, z8_hbm, ones_hbm,
         part_hbm, cntp_hbm,
         agg_sh, cnt_sh, idxs_v, idxd_v, rows_v, ones_v, sem) = refs
    else:
        (x_hbm, src_hbm, dst_hbm, z128_hbm,
         part_hbm,
         agg_sh, idxs_v, idxd_v, rows_v, sem) = refs

    c = lax.axis_index("c")
    s = lax.axis_index("s")

    pltpu.sync_copy(z128_hbm, rows_v)
    for off, n in CHUNKS:
        pltpu.sync_copy(rows_v.at[pl.ds(0, n)],
                        agg_sh.at[pl.ds(s * RPS + off, n)])
    if with_count:
        pltpu.sync_copy(z8_hbm, ones_v)
        for off, n in CHUNKS:
            pltpu.sync_copy(ones_v.at[pl.ds(0, n)],
                            cnt_sh.at[pl.ds(s * RPS + off, n)])
        pltpu.sync_copy(ones_hbm, ones_v)

    mc1 = MCHT - mc0

    @pl.when(c == 0)
    def _():
        pltpu.sync_copy(src_hbm.at[s, pl.ds(0, mc0)], idxs_v.at[pl.ds(0, mc0)])
        pltpu.sync_copy(dst_hbm.at[s, pl.ds(0, mc0)], idxd_v.at[pl.ds(0, mc0)])

    @pl.when(c == 1)
    def _():
        pltpu.sync_copy(src_hbm.at[s, pl.ds(mc0, mc1)], idxs_v.at[pl.ds(0, mc1)])
        pltpu.sync_copy(dst_hbm.at[s, pl.ds(mc0, mc1)], idxd_v.at[pl.ds(0, mc1)])

    nch = lax.select(c == 0, mc0, mc1)

    plsc.subcore_barrier()

    def chunk(j, carry):
        pltpu.async_copy(x_hbm.at[idxs_v.at[j]], rows_v, sem).wait()
        pltpu.sync_copy(rows_v, agg_sh.at[idxd_v.at[j]], add=True)
        if with_count:
            pltpu.sync_copy(ones_v, cnt_sh.at[idxd_v.at[j]], add=True)
        return carry

    lax.fori_loop(0, nch, chunk, 0)

    plsc.subcore_barrier()

    for off, n in CHUNKS:
        r0 = s * RPS + off
        pltpu.sync_copy(agg_sh.at[pl.ds(r0, n)], rows_v.at[pl.ds(0, n)])
        pltpu.sync_copy(rows_v.at[pl.ds(0, n)], part_hbm.at[c, pl.ds(r0, n)])
        if with_count:
            pltpu.sync_copy(cnt_sh.at[pl.ds(r0, n)], ones_v.at[pl.ds(0, n)])
            pltpu.sync_copy(ones_v.at[pl.ds(0, n)],
                            cntp_hbm.at[c, pl.ds(r0, n)])


def _make_sc_agg(with_count):
    mesh = plsc.VectorSubcoreMesh(
        core_axis_name="c", subcore_axis_name="s",
        num_cores=NC, num_subcores=NS)
    mc0 = MC0_CNT if with_count else MC0_PLAIN
    mcx = max(mc0, MCHT - mc0)
    if with_count:
        out_type = (
            jax.ShapeDtypeStruct((NC, N_PAD, D), jnp.float32),
            jax.ShapeDtypeStruct((NC, N_PAD, CW), jnp.float32),
        )
        scratch = [
            pltpu.VMEM_SHARED((N_PAD, D), jnp.float32),
            pltpu.VMEM_SHARED((N_PAD, CW), jnp.float32),
            pltpu.VMEM((mcx, B), jnp.int32),
            pltpu.VMEM((mcx, B), jnp.int32),
            pltpu.VMEM((ZCH, D), jnp.float32),
            pltpu.VMEM((ZCH, CW), jnp.float32),
            pltpu.SemaphoreType.DMA,
        ]
    else:
        out_type = jax.ShapeDtypeStruct((NC, N_PAD, D), jnp.float32)
        scratch = [
            pltpu.VMEM_SHARED((N_PAD, D), jnp.float32),
            pltpu.VMEM((mcx, B), jnp.int32),
            pltpu.VMEM((mcx, B), jnp.int32),
            pltpu.VMEM((ZCH, D), jnp.float32),
            pltpu.SemaphoreType.DMA,
        ]
    return pl.kernel(
        functools.partial(_sc_agg_body, with_count, mc0),
        out_type=out_type, mesh=mesh, scratch_types=scratch,
        compiler_params=pltpu.CompilerParams(use_tc_tiling_on_sc=False),
        name=f"sage_sc_agg_cnt{int(with_count)}")


_R = 1000


def _tc_dense_body(p0, p1, c0, c1, x, wl, wr, b, o):
    agg = p0[0] + p1[0]
    cnt = c0[0][:, 0:1] + c1[0][:, 0:1]
    inv = 1.0 / jnp.maximum(cnt, 1.0)
    g = jnp.dot(agg, wl[...], preferred_element_type=jnp.float32)
    h = jnp.dot(x[...], wr[...], preferred_element_type=jnp.float32)
    o[...] = jnp.maximum(g * inv + h + b[...], 0.0)


def _tc_dense(part, cntp, x, wl, wr, b):
    grid = (N_NODES // _R,)
    p0 = pl.BlockSpec((1, _R, D), lambda i: (0, i, 0))
    p1 = pl.BlockSpec((1, _R, D), lambda i: (1, i, 0))
    c0 = pl.BlockSpec((1, _R, CW), lambda i: (0, i, 0))
    c1 = pl.BlockSpec((1, _R, CW), lambda i: (1, i, 0))
    row = pl.BlockSpec((_R, D), lambda i: (i, 0))
    full = pl.BlockSpec((D, D), lambda i: (0, 0))
    bias = pl.BlockSpec((1, D), lambda i: (0, 0))
    return pl.pallas_call(
        _tc_dense_body,
        grid=grid,
        in_specs=[p0, p1, c0, c1, row, full, full, bias],
        out_specs=row,
        out_shape=jax.ShapeDtypeStruct((N_NODES, D), jnp.float32),
    )(part, part, cntp, cntp, x, wl, wr, b.reshape(1, D))


def kernel(x, edge_index, Wl1, Wr1, b1, Wl2, Wr2, b2):
    n_extra = E_PAD - N_EDGES
    src = edge_index[0].astype(jnp.int32)
    dst = edge_index[1].astype(jnp.int32)
    src = jnp.concatenate([src, jnp.zeros((n_extra,), jnp.int32)])
    pad_dst = N_NODES + (jnp.arange(n_extra, dtype=jnp.int32) % (N_PAD - N_NODES))
    dst = jnp.concatenate([dst, pad_dst])
    src = src.reshape(NS, MCHT, B)
    dst = dst.reshape(NS, MCHT, B)
    x = x.astype(jnp.float32)
    z128 = jnp.zeros((ZCH, D), jnp.float32)
    z8 = jnp.zeros((ZCH, CW), jnp.float32)
    ones = jnp.ones((B, CW), jnp.float32)

    part1, cntp = _make_sc_agg(True)(x, src, dst, z128, z8, ones)
    h = _tc_dense(part1, cntp, x, Wl1, Wr1, b1)
    part2 = _make_sc_agg(False)(h, src, dst, z128)
    out = _tc_dense(part2, cntp, h, Wl2, Wr2, b2)
    return out


# speedup vs baseline: 1.5296x; 1.0369x over previous
import functools

import jax
import jax.numpy as jnp
from jax import lax
from jax.experimental import pallas as pl
from jax.experimental.pallas import tpu as pltpu
from jax.experimental.pallas import tpu_sc as plsc

N_NODES = 10000
N_EDGES = 320000
D = 128

NC = 2
NS = 16
B = 128
BH = 64
MCHT = 158
MC0_CNT = 103
MC0_PLAIN = 110
E_PAD = NS * MCHT * B
N_PAD = 10240
RPS = N_PAD // NS
ZCH = 64
CHUNKS = [(k * ZCH, ZCH) for k in range(RPS // ZCH)]
CW = 8


def _sc_agg_body(with_count, mc0, *refs):
    if with_count:
        (x_hbm, src_hbm, dst_hbm, z128_hbm, z8_hbm, ones_hbm,
         part_hbm, cntp_hbm,
         agg_sh, cnt_sh, idxs_v, idxd_v, rows_v, ones_v,
         sem, sem2, sem3, sem4) = refs
    else:
        (x_hbm, src_hbm, dst_hbm, z128_hbm,
         part_hbm,
         agg_sh, idxs_v, idxd_v, rows_v, ones_v,
         sem, sem2, sem3, sem4) = refs

    c = lax.axis_index("c")
    s = lax.axis_index("s")

    pltpu.sync_copy(z128_hbm, rows_v.at[0])
    for off, n in CHUNKS:
        pltpu.sync_copy(rows_v.at[0], agg_sh.at[pl.ds(s * RPS + off, n)])
    if with_count:
        pltpu.sync_copy(z8_hbm, ones_v)
        for off, n in CHUNKS:
            pltpu.sync_copy(ones_v, cnt_sh.at[pl.ds(s * RPS + off, n)])
        pltpu.sync_copy(ones_hbm, ones_v)

    mc1 = MCHT - mc0

    @pl.when(c == 0)
    def _():
        pltpu.sync_copy(src_hbm.at[s, pl.ds(0, 2 * mc0)],
                        idxs_v.at[pl.ds(0, 2 * mc0)])
        pltpu.sync_copy(dst_hbm.at[s, pl.ds(0, 2 * mc0)],
                        idxd_v.at[pl.ds(0, 2 * mc0)])

    @pl.when(c == 1)
    def _():
        pltpu.sync_copy(src_hbm.at[s, pl.ds(2 * mc0, 2 * mc1)],
                        idxs_v.at[pl.ds(0, 2 * mc1)])
        pltpu.sync_copy(dst_hbm.at[s, pl.ds(2 * mc0, 2 * mc1)],
                        idxd_v.at[pl.ds(0, 2 * mc1)])

    nch = lax.select(c == 0, mc0, mc1)

    plsc.subcore_barrier()

    def chunk(j, carry):
        h0 = 2 * j
        d0 = pltpu.async_copy(x_hbm.at[idxs_v.at[h0]], rows_v.at[0], sem)
        d1 = pltpu.async_copy(x_hbm.at[idxs_v.at[h0 + 1]], rows_v.at[1], sem2)
        d0.wait()
        d1.wait()
        e0 = pltpu.async_copy(
            rows_v.at[0], agg_sh.at[idxd_v.at[h0]], sem3, add=True)
        e1 = pltpu.async_copy(
            rows_v.at[1], agg_sh.at[idxd_v.at[h0 + 1]], sem4, add=True)
        e0.wait()
        e1.wait()
        if with_count:
            pltpu.sync_copy(ones_v, cnt_sh.at[idxd_v.at[h0]], add=True)
            pltpu.sync_copy(ones_v, cnt_sh.at[idxd_v.at[h0 + 1]], add=True)
        return carry

    lax.fori_loop(0, nch, chunk, 0)

    plsc.subcore_barrier()

    for off, n in CHUNKS:
        r0 = s * RPS + off
        pltpu.sync_copy(agg_sh.at[pl.ds(r0, n)], rows_v.at[0])
        pltpu.sync_copy(rows_v.at[0], part_hbm.at[c, pl.ds(r0, n)])
        if with_count:
            pltpu.sync_copy(cnt_sh.at[pl.ds(r0, n)], ones_v)
            pltpu.sync_copy(ones_v, cntp_hbm.at[c, pl.ds(r0, n)])


def _make_sc_agg(with_count):
    mesh = plsc.VectorSubcoreMesh(
        core_axis_name="c", subcore_axis_name="s",
        num_cores=NC, num_subcores=NS)
    mc0 = MC0_CNT if with_count else MC0_PLAIN
    mcx = max(mc0, MCHT - mc0)
    if with_count:
        out_type = (
            jax.ShapeDtypeStruct((NC, N_PAD, D), jnp.float32),
            jax.ShapeDtypeStruct((NC, N_PAD, CW), jnp.float32),
        )
        scratch = [
            pltpu.VMEM_SHARED((N_PAD, D), jnp.float32),
            pltpu.VMEM_SHARED((N_PAD, CW), jnp.float32),
            pltpu.VMEM((2 * mcx, BH), jnp.int32),
            pltpu.VMEM((2 * mcx, BH), jnp.int32),
            pltpu.VMEM((2, BH, D), jnp.float32),
            pltpu.VMEM((ZCH, CW), jnp.float32),
        ] + [pltpu.SemaphoreType.DMA] * 4
    else:
        out_type = jax.ShapeDtypeStruct((NC, N_PAD, D), jnp.float32)
        scratch = [
            pltpu.VMEM_SHARED((N_PAD, D), jnp.float32),
            pltpu.VMEM((2 * mcx, BH), jnp.int32),
            pltpu.VMEM((2 * mcx, BH), jnp.int32),
            pltpu.VMEM((2, BH, D), jnp.float32),
            pltpu.VMEM((ZCH, CW), jnp.float32),
        ] + [pltpu.SemaphoreType.DMA] * 4
    return pl.kernel(
        functools.partial(_sc_agg_body, with_count, mc0),
        out_type=out_type, mesh=mesh, scratch_types=scratch,
        compiler_params=pltpu.CompilerParams(use_tc_tiling_on_sc=False),
        name=f"sage_sc_agg_cnt{int(with_count)}")


_R = 1000


def _tc_dense_body(p0, p1, c0, c1, x, wl, wr, b, o):
    agg = p0[0] + p1[0]
    cnt = c0[0][:, 0:1] + c1[0][:, 0:1]
    inv = 1.0 / jnp.maximum(cnt, 1.0)
    g = jnp.dot(agg, wl[...], preferred_element_type=jnp.float32)
    h = jnp.dot(x[...], wr[...], preferred_element_type=jnp.float32)
    o[...] = jnp.maximum(g * inv + h + b[...], 0.0)


def _tc_dense(part, cntp, x, wl, wr, b):
    grid = (N_NODES // _R,)
    p0 = pl.BlockSpec((1, _R, D), lambda i: (0, i, 0))
    p1 = pl.BlockSpec((1, _R, D), lambda i: (1, i, 0))
    c0 = pl.BlockSpec((1, _R, CW), lambda i: (0, i, 0))
    c1 = pl.BlockSpec((1, _R, CW), lambda i: (1, i, 0))
    row = pl.BlockSpec((_R, D), lambda i: (i, 0))
    full = pl.BlockSpec((D, D), lambda i: (0, 0))
    bias = pl.BlockSpec((1, D), lambda i: (0, 0))
    return pl.pallas_call(
        _tc_dense_body,
        grid=grid,
        in_specs=[p0, p1, c0, c1, row, full, full, bias],
        out_specs=row,
        out_shape=jax.ShapeDtypeStruct((N_NODES, D), jnp.float32),
    )(part, part, cntp, cntp, x, wl, wr, b.reshape(1, D))


def kernel(x, edge_index, Wl1, Wr1, b1, Wl2, Wr2, b2):
    n_extra = E_PAD - N_EDGES
    src = edge_index[0].astype(jnp.int32)
    dst = edge_index[1].astype(jnp.int32)
    src = jnp.concatenate([src, jnp.zeros((n_extra,), jnp.int32)])
    pad_dst = N_NODES + (jnp.arange(n_extra, dtype=jnp.int32) % (N_PAD - N_NODES))
    dst = jnp.concatenate([dst, pad_dst])
    src = src.reshape(NS, 2 * MCHT, BH)
    dst = dst.reshape(NS, 2 * MCHT, BH)
    x = x.astype(jnp.float32)
    z128 = jnp.zeros((ZCH, D), jnp.float32)
    z8 = jnp.zeros((ZCH, CW), jnp.float32)
    ones = jnp.ones((BH, CW), jnp.float32)

    part1, cntp = _make_sc_agg(True)(x, src, dst, z128, z8, ones)
    h = _tc_dense(part1, cntp, x, Wl1, Wr1, b1)
    part2 = _make_sc_agg(False)(h, src, dst, z128)
    out = _tc_dense(part2, cntp, h, Wl2, Wr2, b2)
    return out
